# Initial kernel scaffold; baseline (speedup 1.0000x reference)
import jax
import jax.numpy as jnp
from jax.experimental import pallas as pl


def kernel(x, edge_index, batch, W1, b1, g1, be1, W2, b2, g2, be2, Wl, bl):
    raise NotImplementedError("write your pallas kernel here")


# speedup vs baseline: 6.8632x; 6.8632x over previous
import functools

import jax
import jax.numpy as jnp
from jax import lax
from jax.experimental import pallas as pl
from jax.experimental.pallas import tpu as pltpu
from jax.experimental.pallas import tpu_sc as plsc

N = 10000
D = 128
H = 128
C = 40
G = 64

NC = 2
NS = 16
K = 128
CHUNKS = 80
E_PAD = NC * NS * K * CHUNKS
ROWS_PER_CORE = NS * CHUNKS
NPAD = 10240
ZROWS = NPAD // NS

_mesh = plsc.VectorSubcoreMesh(core_axis_name="c", subcore_axis_name="s")


@functools.partial(
    pl.kernel,
    out_type=jax.ShapeDtypeStruct((NC, NPAD, 16), jnp.float32),
    mesh=_mesh,
    scratch_types=[
        pltpu.VMEM((K,), jnp.int32),
        pltpu.VMEM((K, 16), jnp.float32),
        pltpu.VMEM_SHARED((NPAD, 16), jnp.float32),
    ],
)
def _sc_degree(dst_ref, zeros_ref, out_ref, idx_v, ones_v, acc_sh):
    c = lax.axis_index("c")
    s = lax.axis_index("s")

    def fill(i, _):
        ones_v[i] = jnp.ones((16,), jnp.float32)
        return 0

    lax.fori_loop(0, K, fill, 0)
    pltpu.sync_copy(zeros_ref.at[pl.ds(s * ZROWS, ZROWS)],
                    acc_sh.at[pl.ds(s * ZROWS, ZROWS)])
    plsc.subcore_barrier()

    base_e = (c * ROWS_PER_CORE + s * CHUNKS) * K

    def body(j, _):
        pltpu.sync_copy(dst_ref.at[pl.ds(base_e + j * K, K)], idx_v)
        pltpu.sync_copy(ones_v, acc_sh.at[idx_v], add=True)
        return 0

    lax.fori_loop(0, CHUNKS, body, 0)
    plsc.subcore_barrier()
    pltpu.sync_copy(acc_sh.at[pl.ds(s * ZROWS, ZROWS)],
                    out_ref.at[c, pl.ds(s * ZROWS, ZROWS)])


@functools.partial(
    pl.kernel,
    out_type=jax.ShapeDtypeStruct((NC, NPAD, H), jnp.float32),
    mesh=_mesh,
    scratch_types=[
        pltpu.VMEM((K,), jnp.int32),
        pltpu.VMEM((K,), jnp.int32),
        pltpu.VMEM((K, H), jnp.float32),
        pltpu.VMEM_SHARED((NPAD, H), jnp.float32),
        pltpu.SemaphoreType.DMA,
    ],
)
def _sc_scatter(y_ref, src_ref, dst_ref, zeros_ref, out_ref, idx_s, idx_d, rows,
                acc_sh, sem):
    c = lax.axis_index("c")
    s = lax.axis_index("s")

    pltpu.sync_copy(zeros_ref.at[pl.ds(s * ZROWS, ZROWS)],
                    acc_sh.at[pl.ds(s * ZROWS, ZROWS)])
    plsc.subcore_barrier()

    base_e = (c * ROWS_PER_CORE + s * CHUNKS) * K

    def body(j, _):
        pltpu.sync_copy(src_ref.at[pl.ds(base_e + j * K, K)], idx_s)
        pltpu.sync_copy(dst_ref.at[pl.ds(base_e + j * K, K)], idx_d)
        pltpu.async_copy(y_ref.at[idx_s], rows, sem).wait()
        pltpu.sync_copy(rows, acc_sh.at[idx_d], add=True)
        return 0

    lax.fori_loop(0, CHUNKS, body, 0)
    plsc.subcore_barrier()
    pltpu.sync_copy(acc_sh.at[pl.ds(s * ZROWS, ZROWS)],
                    out_ref.at[c, pl.ds(s * ZROWS, ZROWS)])


BM = 400
GRID = N // BM

_dot = functools.partial(lax.dot_general, precision=lax.Precision.HIGHEST,
                         preferred_element_type=jnp.float32)


def _mm(a, b):
    return _dot(a, b, dimension_numbers=(((1,), (0,)), ((), ())))


def _tc_matmul_scale_body(x_ref, w_ref, d_ref, y_ref):
    y_ref[...] = d_ref[...] * _mm(x_ref[...], w_ref[...])


def _tc_matmul_scale(x, w, dis_col):
    return pl.pallas_call(
        _tc_matmul_scale_body,
        grid=(GRID,),
        in_specs=[
            pl.BlockSpec((BM, D), lambda i: (i, 0)),
            pl.BlockSpec((D, H), lambda i: (0, 0)),
            pl.BlockSpec((BM, 1), lambda i: (i, 0)),
        ],
        out_specs=pl.BlockSpec((BM, H), lambda i: (i, 0)),
        out_shape=jax.ShapeDtypeStruct((N, H), jnp.float32),
    )(x, w, dis_col)


def _tc_post_body(p0_ref, p1_ref, y_ref, d_ref, b_ref, p_ref, st_ref):
    h = d_ref[...] * (p0_ref[0] + p1_ref[0] + y_ref[...]) + b_ref[...]
    pr = jnp.maximum(h, 0.0)
    p_ref[...] = pr

    @pl.when(pl.program_id(0) == 0)
    def _():
        st_ref[...] = jnp.zeros_like(st_ref)

    st_ref[0:1, :] += jnp.sum(pr, axis=0, keepdims=True)
    st_ref[1:2, :] += jnp.sum(pr * pr, axis=0, keepdims=True)


def _tc_post(parts, y, dis_col, b):
    return pl.pallas_call(
        _tc_post_body,
        grid=(GRID,),
        in_specs=[
            pl.BlockSpec((1, BM, H), lambda i: (0, i, 0)),
            pl.BlockSpec((1, BM, H), lambda i: (1, i, 0)),
            pl.BlockSpec((BM, H), lambda i: (i, 0)),
            pl.BlockSpec((BM, 1), lambda i: (i, 0)),
            pl.BlockSpec((1, H), lambda i: (0, 0)),
        ],
        out_specs=[
            pl.BlockSpec((BM, H), lambda i: (i, 0)),
            pl.BlockSpec((8, H), lambda i: (0, 0)),
        ],
        out_shape=[
            jax.ShapeDtypeStruct((N, H), jnp.float32),
            jax.ShapeDtypeStruct((8, H), jnp.float32),
        ],
    )(parts, parts, y, dis_col, b.reshape(1, H))


def _bn_consts(st_ref):
    n = jnp.float32(N)
    mu = st_ref[0:1, :] / n
    var = st_ref[1:2, :] / n - mu * mu
    inv = lax.rsqrt(var + 1e-5)
    return mu, inv


def _tc_norm_matmul_scale_body(p_ref, st_ref, g_ref, be_ref, w_ref, d_ref, y_ref):
    mu, inv = _bn_consts(st_ref)
    xn = (p_ref[...] - mu) * (inv * g_ref[...]) + be_ref[...]
    y_ref[...] = d_ref[...] * _mm(xn, w_ref[...])


def _tc_norm_matmul_scale(p, st, g, be, w, dis_col):
    return pl.pallas_call(
        _tc_norm_matmul_scale_body,
        grid=(GRID,),
        in_specs=[
            pl.BlockSpec((BM, H), lambda i: (i, 0)),
            pl.BlockSpec((8, H), lambda i: (0, 0)),
            pl.BlockSpec((1, H), lambda i: (0, 0)),
            pl.BlockSpec((1, H), lambda i: (0, 0)),
            pl.BlockSpec((H, H), lambda i: (0, 0)),
            pl.BlockSpec((BM, 1), lambda i: (i, 0)),
        ],
        out_specs=pl.BlockSpec((BM, H), lambda i: (i, 0)),
        out_shape=jax.ShapeDtypeStruct((N, H), jnp.float32),
    )(p, st, g.reshape(1, H), be.reshape(1, H), w, dis_col)


def _tc_pool_body(p_ref, st_ref, g_ref, be_ref, b_ref, sums_ref, cnts_ref):
    mu, inv = _bn_consts(st_ref)
    xn = (p_ref[...] - mu) * (inv * g_ref[...]) + be_ref[...]
    ids = lax.broadcasted_iota(jnp.int32, (BM, G), 1)
    sel = (b_ref[...] == ids).astype(jnp.float32)

    @pl.when(pl.program_id(0) == 0)
    def _():
        sums_ref[...] = jnp.zeros_like(sums_ref)
        cnts_ref[...] = jnp.zeros_like(cnts_ref)

    sums_ref[...] += _dot(sel, xn, dimension_numbers=(((0,), (0,)), ((), ())))
    cnts_ref[...] += jnp.broadcast_to(jnp.sum(sel, axis=0)[:, None], (G, H))


def _tc_pool(p, st, g, be, batch_col):
    return pl.pallas_call(
        _tc_pool_body,
        grid=(GRID,),
        in_specs=[
            pl.BlockSpec((BM, H), lambda i: (i, 0)),
            pl.BlockSpec((8, H), lambda i: (0, 0)),
            pl.BlockSpec((1, H), lambda i: (0, 0)),
            pl.BlockSpec((1, H), lambda i: (0, 0)),
            pl.BlockSpec((BM, 1), lambda i: (i, 0)),
        ],
        out_specs=[
            pl.BlockSpec((G, H), lambda i: (0, 0)),
            pl.BlockSpec((G, H), lambda i: (0, 0)),
        ],
        out_shape=[
            jax.ShapeDtypeStruct((G, H), jnp.float32),
            jax.ShapeDtypeStruct((G, H), jnp.float32),
        ],
    )(p, st, g.reshape(1, H), be.reshape(1, H), batch_col)


def _tc_final_body(s_ref, c_ref, w_ref, b_ref, o_ref):
    m = s_ref[...] / jnp.maximum(c_ref[...], 1.0)
    o_ref[...] = _mm(m, w_ref[...]) + b_ref[...]


def _tc_final(sums, cnts, wl, bl):
    return pl.pallas_call(
        _tc_final_body,
        out_shape=jax.ShapeDtypeStruct((G, C), jnp.float32),
    )(sums, cnts, wl, bl.reshape(1, C))


def kernel(x, edge_index, batch, W1, b1, g1, be1, W2, b2, g2, be2, Wl, bl):
    src = edge_index[0]
    dst = edge_index[1]
    pad = E_PAD - src.shape[0]
    src_p = jnp.concatenate([src, jnp.zeros((pad,), jnp.int32)])
    dst_p = jnp.concatenate([dst, jnp.full((pad,), N, jnp.int32)])

    degp = _sc_degree(dst_p, jnp.zeros((NPAD, 16), jnp.float32))
    deg = degp[0, :N, 0:1] + degp[1, :N, 0:1] + 1.0
    dis_col = lax.rsqrt(deg)

    zeros_h = jnp.zeros((NPAD, H), jnp.float32)
    y1 = _tc_matmul_scale(x, W1, dis_col)
    parts1 = _sc_scatter(y1, src_p, dst_p, zeros_h)
    p1, st1 = _tc_post(parts1, y1, dis_col, b1)

    y2 = _tc_norm_matmul_scale(p1, st1, g1, be1, W2, dis_col)
    parts2 = _sc_scatter(y2, src_p, dst_p, zeros_h)
    p2, st2 = _tc_post(parts2, y2, dis_col, b2)

    sums, cnts = _tc_pool(p2, st2, g2, be2, batch.reshape(N, 1))
    return _tc_final(sums, cnts, Wl, bl)


# speedup vs baseline: 25.7498x; 3.7518x over previous
import functools

import jax
import jax.numpy as jnp
from jax import lax
from jax.experimental import pallas as pl
from jax.experimental.pallas import tpu as pltpu
from jax.experimental.pallas import tpu_sc as plsc

N = 10000
D = 128
H = 128
C = 40
G = 64

NC = 2
NS = 16
K = 128
CHUNKS = 80
E_PAD = NC * NS * K * CHUNKS
ROWS_PER_CORE = NS * CHUNKS
NPAD = 10240
ZROWS = NPAD // NS

_mesh = plsc.VectorSubcoreMesh(core_axis_name="c", subcore_axis_name="s")


@functools.partial(
    pl.kernel,
    out_type=jax.ShapeDtypeStruct((NC, NPAD, 16), jnp.float32),
    mesh=_mesh,
    scratch_types=[
        [pltpu.VMEM((K,), jnp.int32) for _ in range(4)],
        pltpu.VMEM((K, 16), jnp.float32),
        pltpu.VMEM_SHARED((NPAD, 16), jnp.float32),
        [pltpu.SemaphoreType.DMA for _ in range(4)],
    ],
)
def _sc_degree(dst_ref, zeros_ref, out_ref, idx_v, ones_v, acc_sh, semi):
    c = lax.axis_index("c")
    s = lax.axis_index("s")

    def fill(i, _):
        ones_v[i] = jnp.ones((16,), jnp.float32)
        return 0

    lax.fori_loop(0, K, fill, 0)
    pltpu.sync_copy(zeros_ref.at[pl.ds(s * ZROWS, ZROWS)],
                    acc_sh.at[pl.ds(s * ZROWS, ZROWS)])
    plsc.subcore_barrier()

    base_e = (c * ROWS_PER_CORE + s * CHUNKS) * K

    def start_idx(j, sl):
        pltpu.async_copy(dst_ref.at[pl.ds(base_e + j * K, K)], idx_v[sl], semi[sl])

    def wait_idx(j, sl):
        pltpu.make_async_copy(dst_ref.at[pl.ds(base_e + j * K, K)], idx_v[sl],
                              semi[sl]).wait()

    NB = CHUNKS // 4
    for k in range(4):
        start_idx(k, k)

    def body(i, _):
        for k in range(4):
            wait_idx(4 * i + k, k)
            pltpu.sync_copy(ones_v, acc_sh.at[idx_v[k]], add=True)

            @pl.when(i < NB - 1)
            def _():
                start_idx(4 * i + 4 + k, k)

        return 0

    lax.fori_loop(0, NB, body, 0)
    plsc.subcore_barrier()
    pltpu.sync_copy(acc_sh.at[pl.ds(s * ZROWS, ZROWS)],
                    out_ref.at[c, pl.ds(s * ZROWS, ZROWS)])


@functools.partial(
    pl.kernel,
    out_type=jax.ShapeDtypeStruct((NC, NPAD, H), jnp.float32),
    mesh=_mesh,
    scratch_types=[
        [pltpu.VMEM((K,), jnp.int32) for _ in range(4)],
        [pltpu.VMEM((K,), jnp.int32) for _ in range(4)],
        [pltpu.VMEM((K, H), jnp.float32) for _ in range(2)],
        pltpu.VMEM_SHARED((NPAD, H), jnp.float32),
        [pltpu.SemaphoreType.DMA for _ in range(4)],
        [pltpu.SemaphoreType.DMA for _ in range(2)],
    ],
)
def _sc_scatter(y_ref, src_ref, dst_ref, zeros_ref, out_ref, idx_s, idx_d, rows,
                acc_sh, semi, semg):
    c = lax.axis_index("c")
    s = lax.axis_index("s")

    pltpu.sync_copy(zeros_ref.at[pl.ds(s * ZROWS, ZROWS)],
                    acc_sh.at[pl.ds(s * ZROWS, ZROWS)])
    plsc.subcore_barrier()

    base_e = (c * ROWS_PER_CORE + s * CHUNKS) * K

    def start_idx(j, sl):
        pltpu.async_copy(src_ref.at[pl.ds(base_e + j * K, K)], idx_s[sl], semi[sl])
        pltpu.async_copy(dst_ref.at[pl.ds(base_e + j * K, K)], idx_d[sl], semi[sl])

    def wait_idx(j, sl):
        pltpu.make_async_copy(src_ref.at[pl.ds(base_e + j * K, K)], idx_s[sl],
                              semi[sl]).wait()
        pltpu.make_async_copy(dst_ref.at[pl.ds(base_e + j * K, K)], idx_d[sl],
                              semi[sl]).wait()

    def start_gather(sl, r):
        pltpu.async_copy(y_ref.at[idx_s[sl]], rows[r], semg[r])

    def wait_gather(sl, r):
        pltpu.make_async_copy(y_ref.at[idx_s[sl]], rows[r], semg[r]).wait()

    def scatter(sl, r):
        pltpu.sync_copy(rows[r], acc_sh.at[idx_d[sl]], add=True)

    NB = CHUNKS // 4
    start_idx(0, 0)
    start_idx(1, 1)

    def body(i, _):
        j0 = 4 * i
        wait_idx(j0, 0)
        start_gather(0, 0)
        start_idx(j0 + 2, 2)

        @pl.when(i > 0)
        def _():
            wait_gather(3, 1)
            scatter(3, 1)

        wait_idx(j0 + 1, 1)
        start_gather(1, 1)
        start_idx(j0 + 3, 3)
        wait_gather(0, 0)
        scatter(0, 0)
        wait_idx(j0 + 2, 2)
        start_gather(2, 0)

        @pl.when(i < NB - 1)
        def _():
            start_idx(j0 + 4, 0)

        wait_gather(1, 1)
        scatter(1, 1)
        wait_idx(j0 + 3, 3)
        start_gather(3, 1)

        @pl.when(i < NB - 1)
        def _():
            start_idx(j0 + 5, 1)

        wait_gather(2, 0)
        scatter(2, 0)
        return 0

    lax.fori_loop(0, NB, body, 0)
    wait_gather(3, 1)
    scatter(3, 1)
    plsc.subcore_barrier()
    pltpu.sync_copy(acc_sh.at[pl.ds(s * ZROWS, ZROWS)],
                    out_ref.at[c, pl.ds(s * ZROWS, ZROWS)])


BM = 400
GRID = N // BM

_dot = functools.partial(lax.dot_general, precision=lax.Precision.HIGHEST,
                         preferred_element_type=jnp.float32)


def _mm(a, b):
    return _dot(a, b, dimension_numbers=(((1,), (0,)), ((), ())))


def _tc_matmul_scale_body(x_ref, w_ref, d_ref, y_ref):
    y_ref[...] = d_ref[...] * _mm(x_ref[...], w_ref[...])


def _tc_matmul_scale(x, w, dis_col):
    return pl.pallas_call(
        _tc_matmul_scale_body,
        grid=(GRID,),
        in_specs=[
            pl.BlockSpec((BM, D), lambda i: (i, 0)),
            pl.BlockSpec((D, H), lambda i: (0, 0)),
            pl.BlockSpec((BM, 1), lambda i: (i, 0)),
        ],
        out_specs=pl.BlockSpec((BM, H), lambda i: (i, 0)),
        out_shape=jax.ShapeDtypeStruct((N, H), jnp.float32),
    )(x, w, dis_col)


def _tc_post_body(p0_ref, p1_ref, y_ref, d_ref, b_ref, p_ref, st_ref):
    h = d_ref[...] * (p0_ref[0] + p1_ref[0] + y_ref[...]) + b_ref[...]
    pr = jnp.maximum(h, 0.0)
    p_ref[...] = pr

    @pl.when(pl.program_id(0) == 0)
    def _():
        st_ref[...] = jnp.zeros_like(st_ref)

    st_ref[0:1, :] += jnp.sum(pr, axis=0, keepdims=True)
    st_ref[1:2, :] += jnp.sum(pr * pr, axis=0, keepdims=True)


def _tc_post(parts, y, dis_col, b):
    return pl.pallas_call(
        _tc_post_body,
        grid=(GRID,),
        in_specs=[
            pl.BlockSpec((1, BM, H), lambda i: (0, i, 0)),
            pl.BlockSpec((1, BM, H), lambda i: (1, i, 0)),
            pl.BlockSpec((BM, H), lambda i: (i, 0)),
            pl.BlockSpec((BM, 1), lambda i: (i, 0)),
            pl.BlockSpec((1, H), lambda i: (0, 0)),
        ],
        out_specs=[
            pl.BlockSpec((BM, H), lambda i: (i, 0)),
            pl.BlockSpec((8, H), lambda i: (0, 0)),
        ],
        out_shape=[
            jax.ShapeDtypeStruct((N, H), jnp.float32),
            jax.ShapeDtypeStruct((8, H), jnp.float32),
        ],
    )(parts, parts, y, dis_col, b.reshape(1, H))


def _bn_consts(st_ref):
    n = jnp.float32(N)
    mu = st_ref[0:1, :] / n
    var = st_ref[1:2, :] / n - mu * mu
    inv = lax.rsqrt(var + 1e-5)
    return mu, inv


def _tc_norm_matmul_scale_body(p_ref, st_ref, g_ref, be_ref, w_ref, d_ref, y_ref):
    mu, inv = _bn_consts(st_ref)
    xn = (p_ref[...] - mu) * (inv * g_ref[...]) + be_ref[...]
    y_ref[...] = d_ref[...] * _mm(xn, w_ref[...])


def _tc_norm_matmul_scale(p, st, g, be, w, dis_col):
    return pl.pallas_call(
        _tc_norm_matmul_scale_body,
        grid=(GRID,),
        in_specs=[
            pl.BlockSpec((BM, H), lambda i: (i, 0)),
            pl.BlockSpec((8, H), lambda i: (0, 0)),
            pl.BlockSpec((1, H), lambda i: (0, 0)),
            pl.BlockSpec((1, H), lambda i: (0, 0)),
            pl.BlockSpec((H, H), lambda i: (0, 0)),
            pl.BlockSpec((BM, 1), lambda i: (i, 0)),
        ],
        out_specs=pl.BlockSpec((BM, H), lambda i: (i, 0)),
        out_shape=jax.ShapeDtypeStruct((N, H), jnp.float32),
    )(p, st, g.reshape(1, H), be.reshape(1, H), w, dis_col)


def _tc_pool_body(p_ref, st_ref, g_ref, be_ref, b_ref, sums_ref, cnts_ref):
    mu, inv = _bn_consts(st_ref)
    xn = (p_ref[...] - mu) * (inv * g_ref[...]) + be_ref[...]
    ids = lax.broadcasted_iota(jnp.int32, (BM, G), 1)
    sel = (b_ref[...] == ids).astype(jnp.float32)

    @pl.when(pl.program_id(0) == 0)
    def _():
        sums_ref[...] = jnp.zeros_like(sums_ref)
        cnts_ref[...] = jnp.zeros_like(cnts_ref)

    sums_ref[...] += _dot(sel, xn, dimension_numbers=(((0,), (0,)), ((), ())))
    cnts_ref[...] += jnp.broadcast_to(jnp.sum(sel, axis=0)[:, None], (G, H))


def _tc_pool(p, st, g, be, batch_col):
    return pl.pallas_call(
        _tc_pool_body,
        grid=(GRID,),
        in_specs=[
            pl.BlockSpec((BM, H), lambda i: (i, 0)),
            pl.BlockSpec((8, H), lambda i: (0, 0)),
            pl.BlockSpec((1, H), lambda i: (0, 0)),
            pl.BlockSpec((1, H), lambda i: (0, 0)),
            pl.BlockSpec((BM, 1), lambda i: (i, 0)),
        ],
        out_specs=[
            pl.BlockSpec((G, H), lambda i: (0, 0)),
            pl.BlockSpec((G, H), lambda i: (0, 0)),
        ],
        out_shape=[
            jax.ShapeDtypeStruct((G, H), jnp.float32),
            jax.ShapeDtypeStruct((G, H), jnp.float32),
        ],
    )(p, st, g.reshape(1, H), be.reshape(1, H), batch_col)


def _tc_final_body(s_ref, c_ref, w_ref, b_ref, o_ref):
    m = s_ref[...] / jnp.maximum(c_ref[...], 1.0)
    o_ref[...] = _mm(m, w_ref[...]) + b_ref[...]


def _tc_final(sums, cnts, wl, bl):
    return pl.pallas_call(
        _tc_final_body,
        out_shape=jax.ShapeDtypeStruct((G, C), jnp.float32),
    )(sums, cnts, wl, bl.reshape(1, C))


def kernel(x, edge_index, batch, W1, b1, g1, be1, W2, b2, g2, be2, Wl, bl):
    src = edge_index[0]
    dst = edge_index[1]
    pad = E_PAD - src.shape[0]
    ar = jnp.arange(pad, dtype=jnp.int32)
    src_p = jnp.concatenate([src, ar % N])
    dst_p = jnp.concatenate([dst, N + ar % (NPAD - N)])

    degp = _sc_degree(dst_p, jnp.zeros((NPAD, 16), jnp.float32))
    deg = degp[0, :N, 0:1] + degp[1, :N, 0:1] + 1.0
    dis_col = lax.rsqrt(deg)

    zeros_h = jnp.zeros((NPAD, H), jnp.float32)
    y1 = _tc_matmul_scale(x, W1, dis_col)
    parts1 = _sc_scatter(y1, src_p, dst_p, zeros_h)
    p1, st1 = _tc_post(parts1, y1, dis_col, b1)

    y2 = _tc_norm_matmul_scale(p1, st1, g1, be1, W2, dis_col)
    parts2 = _sc_scatter(y2, src_p, dst_p, zeros_h)
    p2, st2 = _tc_post(parts2, y2, dis_col, b2)

    sums, cnts = _tc_pool(p2, st2, g2, be2, batch.reshape(N, 1))
    return _tc_final(sums, cnts, Wl, bl)


# speedup vs baseline: 26.1706x; 1.0163x over previous
import functools

import jax
import jax.numpy as jnp
from jax import lax
from jax.experimental import pallas as pl
from jax.experimental.pallas import tpu as pltpu
from jax.experimental.pallas import tpu_sc as plsc

N = 10000
D = 128
H = 128
C = 40
G = 64

NC = 2
NS = 16
K = 128
CHUNKS = 80
E_PAD = NC * NS * K * CHUNKS
ROWS_PER_CORE = NS * CHUNKS
NPAD = 10240
ZROWS = NPAD // NS

_mesh = plsc.VectorSubcoreMesh(core_axis_name="c", subcore_axis_name="s")


@functools.partial(
    pl.kernel,
    out_type=jax.ShapeDtypeStruct((NC, NPAD, 16), jnp.float32),
    mesh=_mesh,
    scratch_types=[
        [pltpu.VMEM((K,), jnp.int32) for _ in range(4)],
        pltpu.VMEM((K, 16), jnp.float32),
        pltpu.VMEM_SHARED((NPAD, 16), jnp.float32),
        [pltpu.SemaphoreType.DMA for _ in range(4)],
    ],
)
def _sc_degree(dst_ref, zeros_ref, out_ref, idx_v, ones_v, acc_sh, semi):
    c = lax.axis_index("c")
    s = lax.axis_index("s")

    def fill(i, _):
        ones_v[i] = jnp.ones((16,), jnp.float32)
        return 0

    lax.fori_loop(0, K, fill, 0)
    pltpu.sync_copy(zeros_ref.at[pl.ds(s * ZROWS, ZROWS)],
                    acc_sh.at[pl.ds(s * ZROWS, ZROWS)])
    plsc.subcore_barrier()

    base_e = (c * ROWS_PER_CORE + s * CHUNKS) * K

    def start_idx(j, sl):
        pltpu.async_copy(dst_ref.at[pl.ds(base_e + j * K, K)], idx_v[sl], semi[sl])

    def wait_idx(j, sl):
        pltpu.make_async_copy(dst_ref.at[pl.ds(base_e + j * K, K)], idx_v[sl],
                              semi[sl]).wait()

    NB = CHUNKS // 4
    for k in range(4):
        start_idx(k, k)

    def body(i, _):
        for k in range(4):
            wait_idx(4 * i + k, k)
            pltpu.sync_copy(ones_v, acc_sh.at[idx_v[k]], add=True)

            @pl.when(i < NB - 1)
            def _():
                start_idx(4 * i + 4 + k, k)

        return 0

    lax.fori_loop(0, NB, body, 0)
    plsc.subcore_barrier()
    pltpu.sync_copy(acc_sh.at[pl.ds(s * ZROWS, ZROWS)],
                    out_ref.at[c, pl.ds(s * ZROWS, ZROWS)])


@functools.partial(
    pl.kernel,
    out_type=jax.ShapeDtypeStruct((NC, NPAD, H), jnp.float32),
    mesh=_mesh,
    scratch_types=[
        [pltpu.VMEM((K,), jnp.int32) for _ in range(4)],
        [pltpu.VMEM((K,), jnp.int32) for _ in range(4)],
        [pltpu.VMEM((K, H), jnp.float32) for _ in range(2)],
        pltpu.VMEM_SHARED((NPAD, H), jnp.float32),
        [pltpu.SemaphoreType.DMA for _ in range(4)],
        [pltpu.SemaphoreType.DMA for _ in range(2)],
    ],
)
def _sc_scatter(y_ref, src_ref, dst_ref, zeros_ref, out_ref, idx_s, idx_d, rows,
                acc_sh, semi, semg):
    c = lax.axis_index("c")
    s = lax.axis_index("s")

    pltpu.sync_copy(zeros_ref.at[pl.ds(s * ZROWS, ZROWS)],
                    acc_sh.at[pl.ds(s * ZROWS, ZROWS)])
    plsc.subcore_barrier()

    base_e = (c * ROWS_PER_CORE + s * CHUNKS) * K

    def start_idx(j, sl):
        pltpu.async_copy(src_ref.at[pl.ds(base_e + j * K, K)], idx_s[sl], semi[sl])
        pltpu.async_copy(dst_ref.at[pl.ds(base_e + j * K, K)], idx_d[sl], semi[sl])

    def wait_idx(j, sl):
        pltpu.make_async_copy(src_ref.at[pl.ds(base_e + j * K, K)], idx_s[sl],
                              semi[sl]).wait()
        pltpu.make_async_copy(dst_ref.at[pl.ds(base_e + j * K, K)], idx_d[sl],
                              semi[sl]).wait()

    def start_gather(sl, r):
        pltpu.async_copy(y_ref.at[idx_s[sl]], rows[r], semg[r])

    def wait_gather(sl, r):
        pltpu.make_async_copy(y_ref.at[idx_s[sl]], rows[r], semg[r]).wait()

    def scatter(sl, r):
        pltpu.sync_copy(rows[r], acc_sh.at[idx_d[sl]], add=True)

    NB = CHUNKS // 4
    start_idx(0, 0)
    start_idx(1, 1)

    def body(i, _):
        j0 = 4 * i
        wait_idx(j0, 0)
        start_gather(0, 0)
        start_idx(j0 + 2, 2)

        @pl.when(i > 0)
        def _():
            wait_gather(3, 1)
            scatter(3, 1)

        wait_idx(j0 + 1, 1)
        start_gather(1, 1)
        start_idx(j0 + 3, 3)
        wait_gather(0, 0)
        scatter(0, 0)
        wait_idx(j0 + 2, 2)
        start_gather(2, 0)

        @pl.when(i < NB - 1)
        def _():
            start_idx(j0 + 4, 0)

        wait_gather(1, 1)
        scatter(1, 1)
        wait_idx(j0 + 3, 3)
        start_gather(3, 1)

        @pl.when(i < NB - 1)
        def _():
            start_idx(j0 + 5, 1)

        wait_gather(2, 0)
        scatter(2, 0)
        return 0

    lax.fori_loop(0, NB, body, 0)
    wait_gather(3, 1)
    scatter(3, 1)
    plsc.subcore_barrier()
    pltpu.sync_copy(acc_sh.at[pl.ds(s * ZROWS, ZROWS)],
                    out_ref.at[c, pl.ds(s * ZROWS, ZROWS)])


BM = 400
GRID = N // BM

_dot = functools.partial(lax.dot_general, precision=lax.Precision.HIGHEST,
                         preferred_element_type=jnp.float32)


def _mm(a, b):
    return _dot(a, b, dimension_numbers=(((1,), (0,)), ((), ())))


def _tc_matmul_scale_body(x_ref, w_ref, d_ref, y_ref):
    y_ref[...] = d_ref[...] * _mm(x_ref[...], w_ref[...])


def _tc_matmul_scale(x, w, dis_col):
    return pl.pallas_call(
        _tc_matmul_scale_body,
        grid=(GRID,),
        in_specs=[
            pl.BlockSpec((BM, D), lambda i: (i, 0)),
            pl.BlockSpec((D, H), lambda i: (0, 0)),
            pl.BlockSpec((BM, 1), lambda i: (i, 0)),
        ],
        out_specs=pl.BlockSpec((BM, H), lambda i: (i, 0)),
        out_shape=jax.ShapeDtypeStruct((N, H), jnp.float32),
    )(x, w, dis_col)


def _bn_consts(st_ref):
    n = jnp.float32(N)
    mu = st_ref[0:1, :] / n
    var = st_ref[1:2, :] / n - mu * mu
    inv = lax.rsqrt(var + 1e-5)
    return mu, inv


def _row_ix(i):
    return jnp.where(i < GRID, i, i - GRID)


def _part_ix(i):
    return jnp.minimum(i, GRID - 1)


def _tc_layer(parts, y, dis_col, b, g, be, w):
    def body(p0_ref, p1_ref, y_ref, d_ref, b_ref, g_ref, be_ref, w_ref,
             o_ref, p_s, st_s):
        i = pl.program_id(0)

        @pl.when(i == 0)
        def _():
            st_s[...] = jnp.zeros_like(st_s)

        @pl.when(i < GRID)
        def _():
            h = d_ref[...] * (p0_ref[0] + p1_ref[0] + y_ref[...]) + b_ref[...]
            pr = jnp.maximum(h, 0.0)
            p_s[pl.ds(_row_ix(i) * BM, BM), :] = pr
            st_s[0:1, :] += jnp.sum(pr, axis=0, keepdims=True)
            st_s[1:2, :] += jnp.sum(pr * pr, axis=0, keepdims=True)

        @pl.when(i >= GRID)
        def _():
            mu, inv = _bn_consts(st_s)
            pr = p_s[pl.ds(_row_ix(i) * BM, BM), :]
            xn = (pr - mu) * (inv * g_ref[...]) + be_ref[...]
            o_ref[...] = d_ref[...] * _mm(xn, w_ref[...])

    return pl.pallas_call(
        body,
        grid=(2 * GRID,),
        in_specs=[
            pl.BlockSpec((1, BM, H), lambda i: (0, _part_ix(i), 0)),
            pl.BlockSpec((1, BM, H), lambda i: (1, _part_ix(i), 0)),
            pl.BlockSpec((BM, H), lambda i: (_part_ix(i), 0)),
            pl.BlockSpec((BM, 1), lambda i: (_row_ix(i), 0)),
            pl.BlockSpec((1, H), lambda i: (0, 0)),
            pl.BlockSpec((1, H), lambda i: (0, 0)),
            pl.BlockSpec((1, H), lambda i: (0, 0)),
            pl.BlockSpec((H, H), lambda i: (0, 0)),
        ],
        out_specs=pl.BlockSpec((BM, H), lambda i: (_row_ix(i), 0)),
        out_shape=jax.ShapeDtypeStruct((N, H), jnp.float32),
        scratch_shapes=[
            pltpu.VMEM((N, H), jnp.float32),
            pltpu.VMEM((8, H), jnp.float32),
        ],
    )(parts, parts, y, dis_col, b.reshape(1, H), g.reshape(1, H),
      be.reshape(1, H), w)


def _tc_tail(parts, y, dis_col, b, g, be, batch_col, wl, bl):
    def body(p0_ref, p1_ref, y_ref, d_ref, bt_ref, b_ref, g_ref, be_ref,
             wl_ref, bl_ref, o_ref, p_s, st_s, sums_s, cnts_s):
        i = pl.program_id(0)

        @pl.when(i == 0)
        def _():
            st_s[...] = jnp.zeros_like(st_s)
            sums_s[...] = jnp.zeros_like(sums_s)
            cnts_s[...] = jnp.zeros_like(cnts_s)

        @pl.when(i < GRID)
        def _():
            h = d_ref[...] * (p0_ref[0] + p1_ref[0] + y_ref[...]) + b_ref[...]
            pr = jnp.maximum(h, 0.0)
            p_s[pl.ds(_row_ix(i) * BM, BM), :] = pr
            st_s[0:1, :] += jnp.sum(pr, axis=0, keepdims=True)
            st_s[1:2, :] += jnp.sum(pr * pr, axis=0, keepdims=True)

        @pl.when(i >= GRID)
        def _():
            mu, inv = _bn_consts(st_s)
            pr = p_s[pl.ds(_row_ix(i) * BM, BM), :]
            xn = (pr - mu) * (inv * g_ref[...]) + be_ref[...]
            ids = lax.broadcasted_iota(jnp.int32, (BM, G), 1)
            sel = (bt_ref[...] == ids).astype(jnp.float32)
            sums_s[...] += _dot(sel, xn, dimension_numbers=(((0,), (0,)), ((), ())))
            cnts_s[...] += jnp.broadcast_to(jnp.sum(sel, axis=0)[:, None], (G, H))

        @pl.when(i == 2 * GRID - 1)
        def _():
            m = sums_s[...] / jnp.maximum(cnts_s[...], 1.0)
            o_ref[...] = _mm(m, wl_ref[...]) + bl_ref[...]

    return pl.pallas_call(
        body,
        grid=(2 * GRID,),
        in_specs=[
            pl.BlockSpec((1, BM, H), lambda i: (0, _part_ix(i), 0)),
            pl.BlockSpec((1, BM, H), lambda i: (1, _part_ix(i), 0)),
            pl.BlockSpec((BM, H), lambda i: (_part_ix(i), 0)),
            pl.BlockSpec((BM, 1), lambda i: (_row_ix(i), 0)),
            pl.BlockSpec((BM, 1), lambda i: (_row_ix(i), 0)),
            pl.BlockSpec((1, H), lambda i: (0, 0)),
            pl.BlockSpec((1, H), lambda i: (0, 0)),
            pl.BlockSpec((1, H), lambda i: (0, 0)),
            pl.BlockSpec((H, C), lambda i: (0, 0)),
            pl.BlockSpec((1, C), lambda i: (0, 0)),
        ],
        out_specs=pl.BlockSpec((G, C), lambda i: (0, 0)),
        out_shape=jax.ShapeDtypeStruct((G, C), jnp.float32),
        scratch_shapes=[
            pltpu.VMEM((N, H), jnp.float32),
            pltpu.VMEM((8, H), jnp.float32),
            pltpu.VMEM((G, H), jnp.float32),
            pltpu.VMEM((G, H), jnp.float32),
        ],
    )(parts, parts, y, dis_col, batch_col, b.reshape(1, H), g.reshape(1, H),
      be.reshape(1, H), wl, bl.reshape(1, C))


def kernel(x, edge_index, batch, W1, b1, g1, be1, W2, b2, g2, be2, Wl, bl):
    src = edge_index[0]
    dst = edge_index[1]
    pad = E_PAD - src.shape[0]
    ar = jnp.arange(pad, dtype=jnp.int32)
    src_p = jnp.concatenate([src, ar % N])
    dst_p = jnp.concatenate([dst, N + ar % (NPAD - N)])

    degp = _sc_degree(dst_p, jnp.zeros((NPAD, 16), jnp.float32))
    deg = degp[0, :N, 0:1] + degp[1, :N, 0:1] + 1.0
    dis_col = lax.rsqrt(deg)

    zeros_h = jnp.zeros((NPAD, H), jnp.float32)
    y1 = _tc_matmul_scale(x, W1, dis_col)
    parts1 = _sc_scatter(y1, src_p, dst_p, zeros_h)
    y2 = _tc_layer(parts1, y1, dis_col, b1, g1, be1, W2)
    parts2 = _sc_scatter(y2, src_p, dst_p, zeros_h)
    return _tc_tail(parts2, y2, dis_col, b2, g2, be2, batch.reshape(N, 1),
                    Wl, bl)
